# Initial kernel scaffold; baseline (speedup 1.0000x reference)
import jax
import jax.numpy as jnp
from jax.experimental import pallas as pl


def kernel(data, segment_ids):
    raise NotImplementedError("write your pallas kernel here")


# speedup vs baseline: 2.6509x; 2.6509x over previous
import jax
import jax.numpy as jnp
from jax import lax
from jax.experimental import pallas as pl
from jax.experimental.pallas import tpu as pltpu
from jax.experimental.pallas import tpu_sc as plsc
import functools

N = 320000
D = 128
S = 10000

NC = 2
NS = 16
DC = D // NC
ROWS_PER_TILE = N // NS
SEG_PER_TILE = S // NS
CHUNK = 80
NCHUNK = ROWS_PER_TILE // CHUNK


def _make_kernel():
    mesh = plsc.VectorSubcoreMesh(core_axis_name="c", subcore_axis_name="s")

    @functools.partial(
        pl.kernel,
        out_type=jax.ShapeDtypeStruct((S, D), jnp.float32),
        mesh=mesh,
        scratch_types=[
            pltpu.VMEM((CHUNK,), jnp.int32),
            pltpu.VMEM((CHUNK, DC), jnp.float32),
            pltpu.VMEM_SHARED((S, DC), jnp.float32),
        ],
        compiler_params=pltpu.CompilerParams(use_tc_tiling_on_sc=False),
    )
    def seg_sum(data_hbm, seg_hbm, zeros_hbm, out_hbm, idx_v, rows_v, acc_sh):
        c = lax.axis_index("c")
        s = lax.axis_index("s")

        pltpu.sync_copy(zeros_hbm, acc_sh.at[pl.ds(s * SEG_PER_TILE, SEG_PER_TILE)])
        plsc.subcore_barrier()

        row0 = s * ROWS_PER_TILE
        col0 = c * DC

        def body(i, carry):
            r = row0 + i * CHUNK
            pltpu.sync_copy(seg_hbm.at[pl.ds(r, CHUNK)], idx_v)
            pltpu.sync_copy(data_hbm.at[pl.ds(r, CHUNK), pl.ds(col0, DC)], rows_v)
            pltpu.sync_copy(rows_v, acc_sh.at[idx_v], add=True)
            return carry

        lax.fori_loop(0, NCHUNK, body, 0)
        plsc.subcore_barrier()

        pltpu.sync_copy(
            acc_sh.at[pl.ds(s * SEG_PER_TILE, SEG_PER_TILE)],
            out_hbm.at[pl.ds(s * SEG_PER_TILE, SEG_PER_TILE), pl.ds(col0, DC)],
        )

    return seg_sum


_seg_sum = _make_kernel()


def kernel(data, segment_ids):
    ids = segment_ids.astype(jnp.int32)
    zeros = jnp.zeros((SEG_PER_TILE, DC), jnp.float32)
    return _seg_sum(data, ids, zeros)


# speedup vs baseline: 7.8283x; 2.9530x over previous
import jax
import jax.numpy as jnp
from jax import lax
from jax.experimental import pallas as pl
from jax.experimental.pallas import tpu as pltpu
from jax.experimental.pallas import tpu_sc as plsc
import functools

N = 320000
D = 128
S = 10000

NC = 2
NS = 16
DC = D // NC
ROWS_PER_TILE = N // NS
SEG_PER_TILE = S // NS
SUB = 100
NSUB = 5
BLK = SUB * NSUB
NBLK = ROWS_PER_TILE // BLK


def _make_kernel():
    mesh = plsc.VectorSubcoreMesh(core_axis_name="c", subcore_axis_name="s")

    @functools.partial(
        pl.kernel,
        out_type=jax.ShapeDtypeStruct((S, D), jnp.float32),
        mesh=mesh,
        scratch_types=[
            pltpu.VMEM((2, NSUB, SUB), jnp.int32),
            pltpu.VMEM((2, BLK, DC), jnp.float32),
            pltpu.VMEM_SHARED((S, DC), jnp.float32),
            pltpu.SemaphoreType.DMA((2,)),
            pltpu.SemaphoreType.DMA((2,)),
            pltpu.SemaphoreType.DMA,
        ],
        compiler_params=pltpu.CompilerParams(use_tc_tiling_on_sc=False),
    )
    def seg_sum(data_hbm, seg_hbm, zeros_hbm, out_hbm,
                idx_v, rows_v, acc_sh, sem_i, sem_d, sem_s):
        c = lax.axis_index("c")
        s = lax.axis_index("s")

        pltpu.sync_copy(zeros_hbm, acc_sh.at[pl.ds(s * SEG_PER_TILE, SEG_PER_TILE)])
        plsc.subcore_barrier()

        row0 = s * ROWS_PER_TILE
        col0 = c * DC

        def start_load(g, b):
            r = row0 + g * BLK
            pltpu.async_copy(
                seg_hbm.at[pl.ds(r // SUB, NSUB)], idx_v.at[b], sem_i.at[b])
            pltpu.async_copy(
                data_hbm.at[pl.ds(r, BLK), pl.ds(col0, DC)], rows_v.at[b],
                sem_d.at[b])

        def wait_load(b):
            pltpu.make_async_copy(
                seg_hbm.at[pl.ds(0, NSUB)], idx_v.at[b], sem_i.at[b]).wait()
            pltpu.make_async_copy(
                data_hbm.at[pl.ds(0, BLK), pl.ds(col0, DC)], rows_v.at[b],
                sem_d.at[b]).wait()

        start_load(0, 0)

        def body(g, carry):
            b = lax.rem(g, 2)

            @pl.when(g + 1 < NBLK)
            def _():
                start_load(g + 1, 1 - b)

            wait_load(b)
            descs = [
                pltpu.async_copy(
                    rows_v.at[b, pl.ds(j * SUB, SUB)],
                    acc_sh.at[idx_v.at[b, j]], sem_s, add=True)
                for j in range(NSUB)
            ]
            for d in descs:
                d.wait()
            return carry

        lax.fori_loop(0, NBLK, body, 0)
        plsc.subcore_barrier()

        pltpu.sync_copy(
            acc_sh.at[pl.ds(s * SEG_PER_TILE, SEG_PER_TILE)],
            out_hbm.at[pl.ds(s * SEG_PER_TILE, SEG_PER_TILE), pl.ds(col0, DC)],
        )

    return seg_sum


_seg_sum = _make_kernel()


def kernel(data, segment_ids):
    ids = segment_ids.astype(jnp.int32).reshape(N // SUB, SUB)
    zeros = jnp.zeros((SEG_PER_TILE, DC), jnp.float32)
    return _seg_sum(data, ids, zeros)


# speedup vs baseline: 8.0153x; 1.0239x over previous
import jax
import jax.numpy as jnp
from jax import lax
from jax.experimental import pallas as pl
from jax.experimental.pallas import tpu as pltpu
from jax.experimental.pallas import tpu_sc as plsc
import functools

N = 320000
D = 128
S = 10000

NC = 2
NS = 16
DC = D // NC
ROWS_PER_TILE = N // NS
SEG_PER_TILE = S // NS
SUB = 125
NSUB = 5
BLK = SUB * NSUB
NBLK = ROWS_PER_TILE // BLK


def _make_kernel():
    mesh = plsc.VectorSubcoreMesh(core_axis_name="c", subcore_axis_name="s")

    @functools.partial(
        pl.kernel,
        out_type=jax.ShapeDtypeStruct((S, D), jnp.float32),
        mesh=mesh,
        scratch_types=[
            pltpu.VMEM((2, NSUB, SUB), jnp.int32),
            pltpu.VMEM((2, BLK, DC), jnp.float32),
            pltpu.VMEM_SHARED((S, DC), jnp.float32),
            pltpu.SemaphoreType.DMA((2,)),
            pltpu.SemaphoreType.DMA((2,)),
            pltpu.SemaphoreType.DMA,
        ],
        compiler_params=pltpu.CompilerParams(use_tc_tiling_on_sc=False),
    )
    def seg_sum(data_hbm, seg_hbm, zeros_hbm, out_hbm,
                idx_v, rows_v, acc_sh, sem_i, sem_d, sem_s):
        c = lax.axis_index("c")
        s = lax.axis_index("s")
        row0 = s * ROWS_PER_TILE
        col0 = c * DC

        def start_load(g, b):
            r = row0 + g * BLK
            pltpu.async_copy(
                seg_hbm.at[pl.ds(r // SUB, NSUB)], idx_v.at[b], sem_i.at[b])
            pltpu.async_copy(
                data_hbm.at[pl.ds(r, BLK), pl.ds(col0, DC)], rows_v.at[b],
                sem_d.at[b])

        def wait_load(b):
            pltpu.make_async_copy(
                seg_hbm.at[pl.ds(0, NSUB)], idx_v.at[b], sem_i.at[b]).wait()
            pltpu.make_async_copy(
                data_hbm.at[pl.ds(0, BLK), pl.ds(col0, DC)], rows_v.at[b],
                sem_d.at[b]).wait()

        start_load(0, 0)
        pltpu.sync_copy(zeros_hbm, acc_sh.at[pl.ds(s * SEG_PER_TILE, SEG_PER_TILE)])
        plsc.subcore_barrier()

        def body(g, carry):
            b = lax.rem(g, 2)

            @pl.when(g + 1 < NBLK)
            def _():
                start_load(g + 1, 1 - b)

            wait_load(b)
            descs = [
                pltpu.async_copy(
                    rows_v.at[b, pl.ds(j * SUB, SUB)],
                    acc_sh.at[idx_v.at[b, j]], sem_s, add=True)
                for j in range(NSUB)
            ]
            for d in descs:
                d.wait()
            return carry

        lax.fori_loop(0, NBLK, body, 0)
        plsc.subcore_barrier()

        pltpu.sync_copy(
            acc_sh.at[pl.ds(s * SEG_PER_TILE, SEG_PER_TILE)],
            out_hbm.at[pl.ds(s * SEG_PER_TILE, SEG_PER_TILE), pl.ds(col0, DC)],
        )

    return seg_sum


_seg_sum = _make_kernel()


def kernel(data, segment_ids):
    ids = segment_ids.astype(jnp.int32).reshape(N // SUB, SUB)
    zeros = jnp.zeros((SEG_PER_TILE, DC), jnp.float32)
    return _seg_sum(data, ids, zeros)


# speedup vs baseline: 8.0307x; 1.0019x over previous
import jax
import jax.numpy as jnp
from jax import lax
from jax.experimental import pallas as pl
from jax.experimental.pallas import tpu as pltpu
from jax.experimental.pallas import tpu_sc as plsc
import functools

N = 320000
D = 128
S = 10000

NC = 2
NS = 16
DC = D // NC
ROWS_PER_TILE = N // NS
SEG_PER_TILE = S // NS
SUB = 100
NSUB = 4
BLK = SUB * NSUB
NBUF = 3
NBLK = ROWS_PER_TILE // BLK


def _make_kernel():
    mesh = plsc.VectorSubcoreMesh(core_axis_name="c", subcore_axis_name="s")

    @functools.partial(
        pl.kernel,
        out_type=jax.ShapeDtypeStruct((S, D), jnp.float32),
        mesh=mesh,
        scratch_types=[
            pltpu.VMEM((NBUF, NSUB, SUB), jnp.int32),
            pltpu.VMEM((NBUF, BLK, DC), jnp.float32),
            pltpu.VMEM_SHARED((S, DC), jnp.float32),
            pltpu.SemaphoreType.DMA((NBUF,)),
            pltpu.SemaphoreType.DMA((NBUF,)),
            pltpu.SemaphoreType.DMA,
        ],
        compiler_params=pltpu.CompilerParams(use_tc_tiling_on_sc=False),
    )
    def seg_sum(data_hbm, seg_hbm, zeros_hbm, out_hbm,
                idx_v, rows_v, acc_sh, sem_i, sem_d, sem_s):
        c = lax.axis_index("c")
        s = lax.axis_index("s")
        row0 = s * ROWS_PER_TILE
        col0 = c * DC

        def start_load(g, b):
            r = row0 + g * BLK
            pltpu.async_copy(
                seg_hbm.at[pl.ds(r // SUB, NSUB)], idx_v.at[b], sem_i.at[b])
            pltpu.async_copy(
                data_hbm.at[pl.ds(r, BLK), pl.ds(col0, DC)], rows_v.at[b],
                sem_d.at[b])

        def wait_load(b):
            pltpu.make_async_copy(
                seg_hbm.at[pl.ds(0, NSUB)], idx_v.at[b], sem_i.at[b]).wait()
            pltpu.make_async_copy(
                data_hbm.at[pl.ds(0, BLK), pl.ds(col0, DC)], rows_v.at[b],
                sem_d.at[b]).wait()

        start_load(0, 0)
        start_load(1, 1)
        pltpu.sync_copy(zeros_hbm, acc_sh.at[pl.ds(s * SEG_PER_TILE, SEG_PER_TILE)])
        plsc.subcore_barrier()

        def body(g, carry):
            b = lax.rem(g, NBUF)

            @pl.when(g + 2 < NBLK)
            def _():
                start_load(g + 2, lax.rem(g + 2, NBUF))

            wait_load(b)
            descs = [
                pltpu.async_copy(
                    rows_v.at[b, pl.ds(j * SUB, SUB)],
                    acc_sh.at[idx_v.at[b, j]], sem_s, add=True)
                for j in range(NSUB)
            ]
            for d in descs:
                d.wait()
            return carry

        lax.fori_loop(0, NBLK, body, 0)
        plsc.subcore_barrier()

        pltpu.sync_copy(
            acc_sh.at[pl.ds(s * SEG_PER_TILE, SEG_PER_TILE)],
            out_hbm.at[pl.ds(s * SEG_PER_TILE, SEG_PER_TILE), pl.ds(col0, DC)],
        )

    return seg_sum


_seg_sum = _make_kernel()


def kernel(data, segment_ids):
    ids = segment_ids.astype(jnp.int32).reshape(N // SUB, SUB)
    zeros = jnp.zeros((SEG_PER_TILE, DC), jnp.float32)
    return _seg_sum(data, ids, zeros)
